# baseline (speedup 1.0000x reference)
import functools

import jax
import jax.numpy as jnp
from jax import lax
from jax.experimental import pallas as pl
from jax.experimental.pallas import tpu as pltpu
from jax.experimental.pallas import tpu_sc as plsc

_INFO = plsc.get_sparse_core_info()
_NC = _INFO.num_cores
_NS = _INFO.num_subcores
_LANES = _INFO.num_lanes
_NW = _NC * _NS

_CHUNK = 128
_RBLK = 8000


def _flatten_body(in_ref, out_ref):
    embed = in_ref.shape[1]
    grp = 128 // embed
    x = in_ref[...].reshape(in_ref.shape[0] // grp, grp, embed)
    for k in range(grp):
        out_ref[:, k * embed:(k + 1) * embed] = x[:, k, :]


@functools.lru_cache(maxsize=None)
def _make_flatten(vocab, embed):
    nblk = vocab // _RBLK
    cols = 128
    rows = _RBLK * embed // cols
    return pl.pallas_call(
        _flatten_body,
        grid=(nblk,),
        in_specs=[pl.BlockSpec((_RBLK, embed), lambda i: (i, 0))],
        out_specs=pl.BlockSpec((rows, cols), lambda i: (i, 0)),
        out_shape=jax.ShapeDtypeStruct((vocab * embed // cols, cols),
                                       jnp.float32),
    )


@functools.lru_cache(maxsize=None)
def _make_sc_kernel(batch, vocab, embed):
    b_per_w = batch // _NW
    n_chunks = b_per_w // _CHUNK
    groups_per_chunk = _CHUNK // _LANES
    grp = 128 // embed
    mesh = plsc.VectorSubcoreMesh(core_axis_name="c", subcore_axis_name="s")

    @functools.partial(
        pl.kernel,
        out_type=jax.ShapeDtypeStruct((batch,), jnp.float32),
        mesh=mesh,
        scratch_types=[
            pltpu.VMEM((b_per_w,), jnp.int32),
            pltpu.VMEM((b_per_w,), jnp.int32),
            pltpu.VMEM((_CHUNK,), jnp.int32),
            pltpu.VMEM((_CHUNK,), jnp.int32),
            pltpu.VMEM((_CHUNK, 128), jnp.float32),
            pltpu.VMEM((_CHUNK, 128), jnp.float32),
            pltpu.VMEM((b_per_w,), jnp.float32),
            pltpu.SemaphoreType.DMA,
        ],
        compiler_params=pltpu.CompilerParams(
            needs_layout_passes=False, use_tc_tiling_on_sc=False),
    )
    def sc_kernel(user_hbm, item_hbm, utab_hbm, itab_hbm, out_hbm,
                  uidx_v, iidx_v, ug_v, ig_v, urows_v, irows_v, out_v, sem):
        wid = lax.axis_index("s") * _NC + lax.axis_index("c")
        base = wid * b_per_w

        pltpu.sync_copy(user_hbm.at[wid], uidx_v)
        pltpu.sync_copy(item_hbm.at[wid], iidx_v)

        lane = lax.iota(jnp.int32, _LANES)

        def chunk_body(c, carry):
            uvs, ivs = [], []
            for g in range(groups_per_chunk):
                off = c * _CHUNK + g * _LANES
                uv = uidx_v[pl.ds(off, _LANES)]
                iv = iidx_v[pl.ds(off, _LANES)]
                ug_v[pl.ds(g * _LANES, _LANES)] = uv // grp
                ig_v[pl.ds(g * _LANES, _LANES)] = iv // grp
                uvs.append((uv % grp) * embed)
                ivs.append((iv % grp) * embed)
            cu = pltpu.async_copy(utab_hbm.at[ug_v], urows_v, sem)
            ci = pltpu.async_copy(itab_hbm.at[ig_v], irows_v, sem)
            cu.wait()
            ci.wait()

            for g in range(groups_per_chunk):
                pos = lane + g * _LANES
                uc0, ic0 = uvs[g], ivs[g]
                acc = jnp.zeros((_LANES,), jnp.float32)
                for d in range(embed):
                    ugv = plsc.load_gather(urows_v, [pos, uc0 + d])
                    igv = plsc.load_gather(irows_v, [pos, ic0 + d])
                    acc = acc + ugv * igv
                out_v[pl.ds(c * _CHUNK + g * _LANES, _LANES)] = acc
            return carry

        lax.fori_loop(0, n_chunks, chunk_body, 0)

        pltpu.sync_copy(out_v, out_hbm.at[pl.ds(base, b_per_w)])

    return sc_kernel


@jax.jit
def kernel(user, item, user_table, item_table):
    batch = user.shape[0]
    vocab, embed = user_table.shape
    b_per_w = batch // _NW
    n_chunks = b_per_w // _CHUNK

    flatten = _make_flatten(vocab, embed)
    uflat = flatten(user_table)
    iflat = flatten(item_table)

    sc = _make_sc_kernel(batch, vocab, embed)
    u = user.astype(jnp.int32).reshape(_NW, b_per_w)
    i = item.astype(jnp.int32).reshape(_NW, b_per_w)
    return sc(u, i, uflat, iflat)


# speedup vs baseline: 1.9662x; 1.9662x over previous
import functools

import jax
import jax.numpy as jnp
from jax import lax
from jax.experimental import pallas as pl
from jax.experimental.pallas import tpu as pltpu
from jax.experimental.pallas import tpu_sc as plsc

_INFO = plsc.get_sparse_core_info()
_NC = _INFO.num_cores
_NS = _INFO.num_subcores
_LANES = _INFO.num_lanes
_NW = _NC * _NS

_CHUNK = 128


@functools.lru_cache(maxsize=None)
def _make_sc_kernel(batch, embed):
    b_per_w = batch // _NW
    n_chunks = b_per_w // _CHUNK
    groups_per_chunk = _CHUNK // _LANES
    mesh = plsc.VectorSubcoreMesh(core_axis_name="c", subcore_axis_name="s")

    @functools.partial(
        pl.kernel,
        out_type=jax.ShapeDtypeStruct((batch,), jnp.float32),
        mesh=mesh,
        scratch_types=[
            pltpu.VMEM((b_per_w,), jnp.int32),
            pltpu.VMEM((b_per_w,), jnp.int32),
            pltpu.VMEM((_CHUNK, embed), jnp.float32),
            pltpu.VMEM((_CHUNK, embed), jnp.float32),
            pltpu.VMEM((b_per_w,), jnp.float32),
            pltpu.SemaphoreType.DMA,
        ],
        compiler_params=pltpu.CompilerParams(needs_layout_passes=False),
    )
    def sc_kernel(user_hbm, item_hbm, utab_hbm, itab_hbm, out_hbm,
                  uidx_v, iidx_v, urows_v, irows_v, out_v, sem):
        wid = lax.axis_index("s") * _NC + lax.axis_index("c")
        base = wid * b_per_w

        pltpu.sync_copy(user_hbm.at[wid], uidx_v)
        pltpu.sync_copy(item_hbm.at[wid], iidx_v)

        lane = lax.iota(jnp.int32, _LANES)

        def chunk_body(c, carry):
            copies = []
            for g in range(groups_per_chunk):
                off = c * _CHUNK + g * _LANES
                uv = uidx_v[pl.ds(off, _LANES)]
                iv = iidx_v[pl.ds(off, _LANES)]
                for k in range(_LANES):
                    dst = pl.ds(g * _LANES + k, 1)
                    copies.append(
                        pltpu.async_copy(utab_hbm.at[pl.ds(uv[k], 1), :],
                                         urows_v.at[dst, :], sem))
                    copies.append(
                        pltpu.async_copy(itab_hbm.at[pl.ds(iv[k], 1), :],
                                         irows_v.at[dst, :], sem))
            for cp in copies:
                cp.wait()

            for g in range(groups_per_chunk):
                rows = lane + g * _LANES
                acc = jnp.zeros((_LANES,), jnp.float32)
                for d in range(embed):
                    col = jnp.full((_LANES,), d, jnp.int32)
                    ug = plsc.load_gather(urows_v, [rows, col])
                    ig = plsc.load_gather(irows_v, [rows, col])
                    acc = acc + ug * ig
                out_v[pl.ds(c * _CHUNK + g * _LANES, _LANES)] = acc
            return carry

        lax.fori_loop(0, n_chunks, chunk_body, 0)

        pltpu.sync_copy(out_v, out_hbm.at[pl.ds(base, b_per_w)])

    return sc_kernel


@jax.jit
def kernel(user, item, user_table, item_table):
    batch = user.shape[0]
    embed = user_table.shape[1]
    sc = _make_sc_kernel(batch, embed)
    u = user.astype(jnp.int32).reshape(_NW, batch // _NW)
    i = item.astype(jnp.int32).reshape(_NW, batch // _NW)
    return sc(u, i, user_table, item_table)
